# baseline (speedup 1.0000x reference)
import functools

import jax
import jax.numpy as jnp
from jax import lax
from jax.experimental import pallas as pl
from jax.experimental.pallas import tpu as pltpu
from jax.experimental.pallas import tpu_sc as plsc

N = 16384
D = 128
K = N // 2
DT = 64

NC = 2
NS = 16
NW = NC * NS
C = N // NS
VPT = C // 16
ROWS_W = K // NW

_GDN = lax.GatherDimensionNumbers(
    offset_dims=(), collapsed_slice_dims=(0,), start_index_map=(0,))


def _vgather16(x, idx):
    return lax.gather(x, idx[:, None], _GDN, (1,),
                      mode=lax.GatherScatterMode.PROMISE_IN_BOUNDS)


def _score_key_body(wp_ref, wg_ref, f_ref, wphy_ref, wgen_ref, o_ref):
    sp = jnp.dot(f_ref[...], wphy_ref[...])
    sg = jnp.dot(f_ref[...], wgen_ref[...])
    comb = wp_ref[0, 0] * sp + wg_ref[0, 0] * sg
    u = lax.bitcast_convert_type(comb, jnp.int32)
    m = lax.shift_right_arithmetic(u, 31)
    key = u ^ jnp.bitwise_not(m | jnp.int32(-2147483648))
    o_ref[...] = lax.bitcast_convert_type(key, jnp.uint32)


def _scores_to_keys(feature, weight_phy, weight_gen, w_phy, w_gen):
    keys = pl.pallas_call(
        _score_key_body,
        in_specs=[
            pl.BlockSpec(memory_space=pltpu.SMEM),
            pl.BlockSpec(memory_space=pltpu.SMEM),
            pl.BlockSpec(memory_space=pltpu.VMEM),
            pl.BlockSpec(memory_space=pltpu.VMEM),
            pl.BlockSpec(memory_space=pltpu.VMEM),
        ],
        out_shape=jax.ShapeDtypeStruct((N, 1), jnp.uint32),
    )(weight_phy.reshape(1, 1), weight_gen.reshape(1, 1), feature, w_phy, w_gen)
    return keys.reshape(N)


def _radix_pass(tid, sh, keys_hbm, src_k, src_v, dst_k, dst_v,
                keys_l, vals_l, hist2d, hist, allhist, startv, countv,
                tmp16, dest2d, sh_hists, sem):
    iota = lax.iota(jnp.int32, 16)
    z16 = jnp.zeros((16,), jnp.int32)
    one16 = jnp.ones((16,), jnp.int32)

    if src_k is None:
        pltpu.sync_copy(keys_hbm.at[pl.ds(tid * C, C)], keys_l)

        def init_vals(i, _):
            vals_l[pl.ds(16 * i, 16)] = C * tid + 16 * i + iota
            return 0
        lax.fori_loop(0, VPT, init_vals, 0)
    else:
        pltpu.sync_copy(src_k.at[pl.ds(tid * C, C)], keys_l)
        pltpu.sync_copy(src_v.at[pl.ds(tid * C, C)], vals_l)

    def hist_body(i, _):
        k = keys_l[pl.ds(16 * i, 16)]
        d = ((k >> sh) & 255).astype(jnp.int32)
        plsc.addupdate_scatter(hist2d, [iota * 256 + d], one16)
        return 0
    lax.fori_loop(0, VPT, hist_body, 0)

    for c in range(16):
        def red_body(r, acc):
            row = hist2d[pl.ds(r * 256 + 16 * c, 16)]
            hist2d[pl.ds(r * 256 + 16 * c, 16)] = z16
            return acc + row
        hist[pl.ds(16 * c, 16)] = lax.fori_loop(0, 16, red_body, z16)

    pltpu.sync_copy(hist, sh_hists.at[pl.ds(tid * 256, 256)])
    plsc.subcore_barrier()
    pltpu.sync_copy(sh_hists, allhist)

    carry = jnp.int32(0)
    for c in range(16):
        def scan_body(t, pt):
            pre, tot = pt
            row = allhist[pl.ds(t * 256 + 16 * c, 16)]
            pre = pre + jnp.where(t < tid, row, 0)
            return pre, tot + row
        pre, tot = lax.fori_loop(0, 16, scan_body, (z16, z16))
        excl = plsc.cumsum(tot) - tot
        startv[pl.ds(16 * c, 16)] = excl + carry + pre
        carry = carry + jnp.sum(tot)

    for c in range(16):
        countv[pl.ds(16 * c, 16)] = z16

    def rank_body(i, _):
        k = keys_l[pl.ds(16 * i, 16)]
        d = ((k >> sh) & 255).astype(jnp.int32)
        skey = d * 16 + iota
        srt = lax.sort(skey, dimension=0)
        d_s = srt >> 4
        lane_s = srt & 15
        prev = _vgather16(d_s, jnp.maximum(iota - 1, 0))
        nxt = _vgather16(d_s, jnp.minimum(iota + 1, 15))
        newgrp = (iota == 0) | (d_s != prev)
        r = iota - plsc.cummax(jnp.where(newgrp, iota, 0))
        oldc = plsc.load_gather(countv, [d_s])
        bases = plsc.load_gather(startv, [d_s])
        pos = bases + oldc + r
        is_last = (iota == 15) | (d_s != nxt)
        plsc.store_scatter(countv, [d_s], oldc + r + 1, mask=is_last)
        plsc.store_scatter(tmp16, [lane_s], pos)
        dest2d[i // 8, pl.ds(16 * (i % 8), 16)] = tmp16[...]
        return 0
    lax.fori_loop(0, VPT, rank_body, 0)

    descs = []
    for j in range(8):
        idx_row = dest2d.at[j]
        if dst_k is not None:
            descs.append(pltpu.async_copy(
                keys_l.at[pl.ds(128 * j, 128)], dst_k.at[idx_row], sem))
        descs.append(pltpu.async_copy(
            vals_l.at[pl.ds(128 * j, 128)], dst_v.at[idx_row], sem))
    for dsc in descs:
        dsc.wait()
    plsc.subcore_barrier()


def _make_sort_gather():
    mesh = plsc.VectorSubcoreMesh(core_axis_name="c", subcore_axis_name="s")

    @functools.partial(
        pl.kernel,
        out_type=jax.ShapeDtypeStruct((K, 128), jnp.float32),
        mesh=mesh,
        compiler_params=pltpu.CompilerParams(needs_layout_passes=False),
        scratch_types=dict(
            keys_l=pltpu.VMEM((C,), jnp.uint32),
            vals_l=pltpu.VMEM((C,), jnp.int32),
            hist2d=pltpu.VMEM((4096,), jnp.int32),
            hist=pltpu.VMEM((256,), jnp.int32),
            allhist=pltpu.VMEM((4096,), jnp.int32),
            startv=pltpu.VMEM((256,), jnp.int32),
            countv=pltpu.VMEM((256,), jnp.int32),
            tmp16=pltpu.VMEM((16,), jnp.int32),
            dest2d=pltpu.VMEM((8, 128), jnp.int32),
            idx2d=pltpu.VMEM((2, 128), jnp.int32),
            rows_v=pltpu.VMEM((ROWS_W, 128), jnp.float32),
            a_keys=pltpu.VMEM_SHARED((N,), jnp.uint32),
            a_vals=pltpu.VMEM_SHARED((N,), jnp.int32),
            b_keys=pltpu.VMEM_SHARED((N,), jnp.uint32),
            b_vals=pltpu.VMEM_SHARED((N,), jnp.int32),
            sh_hists=pltpu.VMEM_SHARED((NS * 256,), jnp.int32),
            sem=pltpu.SemaphoreType.DMA,
        ),
    )
    def sort_gather(keys_hbm, y_hbm, out_hbm, keys_l, vals_l, hist2d, hist,
                    allhist, startv, countv, tmp16, dest2d, idx2d, rows_v,
                    a_keys, a_vals, b_keys, b_vals, sh_hists, sem):
        cid = lax.axis_index("c")
        tid = lax.axis_index("s")
        z16 = jnp.zeros((16,), jnp.int32)

        def zero_body(i, _):
            hist2d[pl.ds(16 * i, 16)] = z16
            return 0
        lax.fori_loop(0, 256, zero_body, 0)

        common = (keys_l, vals_l, hist2d, hist, allhist, startv, countv,
                  tmp16, dest2d, sh_hists, sem)
        _radix_pass(tid, 0, keys_hbm, None, None, b_keys, b_vals, *common)
        _radix_pass(tid, 8, keys_hbm, b_keys, b_vals, a_keys, a_vals, *common)
        _radix_pass(tid, 16, keys_hbm, a_keys, a_vals, b_keys, b_vals, *common)
        _radix_pass(tid, 24, keys_hbm, b_keys, b_vals, None, a_vals, *common)

        w = tid * NC + cid
        base = w * ROWS_W
        pltpu.sync_copy(a_vals.at[pl.ds(base, 128)], idx2d.at[0])
        pltpu.sync_copy(a_vals.at[pl.ds(base + 128, 128)], idx2d.at[1])
        g0 = pltpu.async_copy(y_hbm.at[idx2d.at[0]],
                              rows_v.at[pl.ds(0, 128)], sem)
        g1 = pltpu.async_copy(y_hbm.at[idx2d.at[1]],
                              rows_v.at[pl.ds(128, 128)], sem)
        g0.wait()
        g1.wait()
        pltpu.sync_copy(rows_v, out_hbm.at[pl.ds(base, ROWS_W)])

    return sort_gather


_SORT_GATHER = _make_sort_gather()


def kernel(x, feature, y, weight_phy, weight_gen, w_phy, w_gen):
    keys = _scores_to_keys(feature, weight_phy, weight_gen, w_phy, w_gen)
    y_pad = jnp.pad(y, ((0, 0), (0, 128 - DT)))
    y_sel = _SORT_GATHER(keys, y_pad)[:, :DT]
    return (x, feature, y_sel)


# speedup vs baseline: 1.0110x; 1.0110x over previous
import functools

import jax
import jax.numpy as jnp
from jax import lax
from jax.experimental import pallas as pl
from jax.experimental.pallas import tpu as pltpu
from jax.experimental.pallas import tpu_sc as plsc

N = 16384
D = 128
K = N // 2
DT = 64

NC = 2
NS = 16
NW = NC * NS
C = N // NS
VPT = C // 16
ROWS_W = K // NW

_GDN = lax.GatherDimensionNumbers(
    offset_dims=(), collapsed_slice_dims=(0,), start_index_map=(0,))


def _vgather16(x, idx):
    return lax.gather(x, idx[:, None], _GDN, (1,),
                      mode=lax.GatherScatterMode.PROMISE_IN_BOUNDS)


def _score_key_body(wp_ref, wg_ref, f_ref, wphy_ref, wgen_ref, y_ref,
                    o_ref, ypad_ref):
    sp = jnp.dot(f_ref[...], wphy_ref[...])
    sg = jnp.dot(f_ref[...], wgen_ref[...])
    comb = wp_ref[0, 0] * sp + wg_ref[0, 0] * sg
    u = lax.bitcast_convert_type(comb, jnp.int32)
    m = lax.shift_right_arithmetic(u, 31)
    key = u ^ jnp.bitwise_not(m | jnp.int32(-2147483648))
    o_ref[...] = lax.bitcast_convert_type(key, jnp.uint32)
    ypad_ref[:, :DT] = y_ref[...]


def _scores_to_keys(feature, weight_phy, weight_gen, w_phy, w_gen, y):
    keys, y_pad = pl.pallas_call(
        _score_key_body,
        in_specs=[
            pl.BlockSpec(memory_space=pltpu.SMEM),
            pl.BlockSpec(memory_space=pltpu.SMEM),
            pl.BlockSpec(memory_space=pltpu.VMEM),
            pl.BlockSpec(memory_space=pltpu.VMEM),
            pl.BlockSpec(memory_space=pltpu.VMEM),
            pl.BlockSpec(memory_space=pltpu.VMEM),
        ],
        out_shape=(jax.ShapeDtypeStruct((N, 1), jnp.uint32),
                   jax.ShapeDtypeStruct((N, 128), jnp.float32)),
    )(weight_phy.reshape(1, 1), weight_gen.reshape(1, 1), feature, w_phy,
      w_gen, y)
    return keys.reshape(N), y_pad


def _radix_pass(tid, sh, keys_hbm, src_k, src_v, dst_k, dst_v,
                keys_l, vals_l, hist2d, hist, allhist, startv, countv,
                tmp16, dest2d, sh_hists, sem):
    iota = lax.iota(jnp.int32, 16)
    z16 = jnp.zeros((16,), jnp.int32)
    one16 = jnp.ones((16,), jnp.int32)

    if src_k is None:
        pltpu.sync_copy(keys_hbm.at[pl.ds(tid * C, C)], keys_l)

        def init_vals(i, _):
            vals_l[pl.ds(16 * i, 16)] = C * tid + 16 * i + iota
            return 0
        lax.fori_loop(0, VPT, init_vals, 0)
    else:
        pltpu.sync_copy(src_k.at[pl.ds(tid * C, C)], keys_l)
        pltpu.sync_copy(src_v.at[pl.ds(tid * C, C)], vals_l)

    def hist_body(i, _):
        k = keys_l[pl.ds(16 * i, 16)]
        d = ((k >> sh) & 255).astype(jnp.int32)
        plsc.addupdate_scatter(hist2d, [iota * 256 + d], one16)
        return 0
    lax.fori_loop(0, VPT, hist_body, 0)

    for c in range(16):
        def red_body(r, acc):
            row = hist2d[pl.ds(r * 256 + 16 * c, 16)]
            hist2d[pl.ds(r * 256 + 16 * c, 16)] = z16
            return acc + row
        hist[pl.ds(16 * c, 16)] = lax.fori_loop(0, 16, red_body, z16)

    pltpu.sync_copy(hist, sh_hists.at[pl.ds(tid * 256, 256)])
    plsc.subcore_barrier()
    pltpu.sync_copy(sh_hists, allhist)

    carry = jnp.int32(0)
    for c in range(16):
        def scan_body(t, pt):
            pre, tot = pt
            row = allhist[pl.ds(t * 256 + 16 * c, 16)]
            pre = pre + jnp.where(t < tid, row, 0)
            return pre, tot + row
        pre, tot = lax.fori_loop(0, 16, scan_body, (z16, z16))
        excl = plsc.cumsum(tot) - tot
        startv[pl.ds(16 * c, 16)] = excl + carry + pre
        carry = carry + jnp.sum(tot)

    for c in range(16):
        countv[pl.ds(16 * c, 16)] = z16

    def rank_body(i, _):
        k = keys_l[pl.ds(16 * i, 16)]
        d = ((k >> sh) & 255).astype(jnp.int32)
        skey = d * 16 + iota
        srt = lax.sort(skey, dimension=0)
        d_s = srt >> 4
        lane_s = srt & 15
        prev = _vgather16(d_s, jnp.maximum(iota - 1, 0))
        nxt = _vgather16(d_s, jnp.minimum(iota + 1, 15))
        newgrp = (iota == 0) | (d_s != prev)
        r = iota - plsc.cummax(jnp.where(newgrp, iota, 0))
        oldc = plsc.load_gather(countv, [d_s])
        bases = plsc.load_gather(startv, [d_s])
        pos = bases + oldc + r
        is_last = (iota == 15) | (d_s != nxt)
        plsc.store_scatter(countv, [d_s], oldc + r + 1, mask=is_last)
        plsc.store_scatter(tmp16, [lane_s], pos)
        dest2d[i // 8, pl.ds(16 * (i % 8), 16)] = tmp16[...]
        return 0
    lax.fori_loop(0, VPT, rank_body, 0)

    descs = []
    for j in range(8):
        idx_row = dest2d.at[j]
        if dst_k is not None:
            descs.append(pltpu.async_copy(
                keys_l.at[pl.ds(128 * j, 128)], dst_k.at[idx_row], sem))
        descs.append(pltpu.async_copy(
            vals_l.at[pl.ds(128 * j, 128)], dst_v.at[idx_row], sem))
    for dsc in descs:
        dsc.wait()
    plsc.subcore_barrier()


def _make_sort_gather():
    mesh = plsc.VectorSubcoreMesh(core_axis_name="c", subcore_axis_name="s")

    @functools.partial(
        pl.kernel,
        out_type=jax.ShapeDtypeStruct((K, 128), jnp.float32),
        mesh=mesh,
        compiler_params=pltpu.CompilerParams(needs_layout_passes=False),
        scratch_types=dict(
            keys_l=pltpu.VMEM((C,), jnp.uint32),
            vals_l=pltpu.VMEM((C,), jnp.int32),
            hist2d=pltpu.VMEM((4096,), jnp.int32),
            hist=pltpu.VMEM((256,), jnp.int32),
            allhist=pltpu.VMEM((4096,), jnp.int32),
            startv=pltpu.VMEM((256,), jnp.int32),
            countv=pltpu.VMEM((256,), jnp.int32),
            tmp16=pltpu.VMEM((16,), jnp.int32),
            dest2d=pltpu.VMEM((8, 128), jnp.int32),
            idx2d=pltpu.VMEM((2, 128), jnp.int32),
            rows_v=pltpu.VMEM((ROWS_W, 128), jnp.float32),
            a_keys=pltpu.VMEM_SHARED((N,), jnp.uint32),
            a_vals=pltpu.VMEM_SHARED((N,), jnp.int32),
            b_keys=pltpu.VMEM_SHARED((N,), jnp.uint32),
            b_vals=pltpu.VMEM_SHARED((N,), jnp.int32),
            sh_hists=pltpu.VMEM_SHARED((NS * 256,), jnp.int32),
            sem=pltpu.SemaphoreType.DMA,
        ),
    )
    def sort_gather(keys_hbm, y_hbm, out_hbm, keys_l, vals_l, hist2d, hist,
                    allhist, startv, countv, tmp16, dest2d, idx2d, rows_v,
                    a_keys, a_vals, b_keys, b_vals, sh_hists, sem):
        cid = lax.axis_index("c")
        tid = lax.axis_index("s")
        z16 = jnp.zeros((16,), jnp.int32)

        def zero_body(i, _):
            hist2d[pl.ds(16 * i, 16)] = z16
            return 0
        lax.fori_loop(0, 256, zero_body, 0)

        common = (keys_l, vals_l, hist2d, hist, allhist, startv, countv,
                  tmp16, dest2d, sh_hists, sem)
        _radix_pass(tid, 0, keys_hbm, None, None, b_keys, b_vals, *common)
        _radix_pass(tid, 8, keys_hbm, b_keys, b_vals, a_keys, a_vals, *common)
        _radix_pass(tid, 16, keys_hbm, a_keys, a_vals, b_keys, b_vals, *common)
        _radix_pass(tid, 24, keys_hbm, b_keys, b_vals, None, a_vals, *common)

        w = tid * NC + cid
        base = w * ROWS_W
        pltpu.sync_copy(a_vals.at[pl.ds(base, 128)], idx2d.at[0])
        pltpu.sync_copy(a_vals.at[pl.ds(base + 128, 128)], idx2d.at[1])
        g0 = pltpu.async_copy(y_hbm.at[idx2d.at[0]],
                              rows_v.at[pl.ds(0, 128)], sem)
        g1 = pltpu.async_copy(y_hbm.at[idx2d.at[1]],
                              rows_v.at[pl.ds(128, 128)], sem)
        g0.wait()
        g1.wait()
        pltpu.sync_copy(rows_v, out_hbm.at[pl.ds(base, ROWS_W)])

    return sort_gather


_SORT_GATHER = _make_sort_gather()


def kernel(x, feature, y, weight_phy, weight_gen, w_phy, w_gen):
    keys, y_pad = _scores_to_keys(feature, weight_phy, weight_gen, w_phy,
                                  w_gen, y)
    y_sel = _SORT_GATHER(keys, y_pad)[:, :DT]
    return (x, feature, y_sel)


# speedup vs baseline: 1.0178x; 1.0067x over previous
import functools

import jax
import jax.numpy as jnp
from jax import lax
from jax.experimental import pallas as pl
from jax.experimental.pallas import tpu as pltpu
from jax.experimental.pallas import tpu_sc as plsc

N = 16384
D = 128
K = N // 2
DT = 64

NC = 2
NS = 16
NW = NC * NS
C = N // NS
VPT = C // 16
ROWS_W = K // NW

_GDN = lax.GatherDimensionNumbers(
    offset_dims=(), collapsed_slice_dims=(0,), start_index_map=(0,))


def _vgather16(x, idx):
    return lax.gather(x, idx[:, None], _GDN, (1,),
                      mode=lax.GatherScatterMode.PROMISE_IN_BOUNDS)


def _score_key_body(wp_ref, wg_ref, f_ref, wphy_ref, wgen_ref, y_ref,
                    o_ref, ypad_ref):
    sp = jnp.dot(f_ref[...], wphy_ref[...])
    sg = jnp.dot(f_ref[...], wgen_ref[...])
    comb = wp_ref[0, 0] * sp + wg_ref[0, 0] * sg
    u = lax.bitcast_convert_type(comb, jnp.int32)
    m = lax.shift_right_arithmetic(u, 31)
    key = u ^ jnp.bitwise_not(m | jnp.int32(-2147483648))
    o_ref[...] = lax.bitcast_convert_type(key, jnp.uint32)
    ypad_ref[:, :DT] = y_ref[...]


def _scores_to_keys(feature, weight_phy, weight_gen, w_phy, w_gen, y):
    keys, y_pad = pl.pallas_call(
        _score_key_body,
        in_specs=[
            pl.BlockSpec(memory_space=pltpu.SMEM),
            pl.BlockSpec(memory_space=pltpu.SMEM),
            pl.BlockSpec(memory_space=pltpu.VMEM),
            pl.BlockSpec(memory_space=pltpu.VMEM),
            pl.BlockSpec(memory_space=pltpu.VMEM),
            pl.BlockSpec(memory_space=pltpu.VMEM),
        ],
        out_shape=(jax.ShapeDtypeStruct((N, 1), jnp.uint32),
                   jax.ShapeDtypeStruct((N, 128), jnp.float32)),
    )(weight_phy.reshape(1, 1), weight_gen.reshape(1, 1), feature, w_phy,
      w_gen, y)
    return keys.reshape(N), y_pad


def _radix_pass(tid, sh, keys_hbm, src_k, src_v, dst_k, dst_v,
                keys_l, vals_l, hist2d, hist, blk, preblk, totv, prev_own,
                startv, countv, tmp16, dest2d, sh_hists, sh_pre, sh_tot, sem):
    iota = lax.iota(jnp.int32, 16)
    z16 = jnp.zeros((16,), jnp.int32)
    one16 = jnp.ones((16,), jnp.int32)

    if src_k is None:
        pltpu.sync_copy(keys_hbm.at[pl.ds(tid * C, C)], keys_l)

        def init_vals(i, _):
            vals_l[pl.ds(16 * i, 16)] = C * tid + 16 * i + iota
            return 0
        lax.fori_loop(0, VPT, init_vals, 0)
    else:
        d0 = pltpu.async_copy(src_k.at[pl.ds(tid * C, C)], keys_l, sem)
        d1 = pltpu.async_copy(src_v.at[pl.ds(tid * C, C)], vals_l, sem)
        d0.wait()
        d1.wait()

    def hist_body(i, _):
        k = keys_l[pl.ds(16 * i, 16)]
        d = ((k >> sh) & 255).astype(jnp.int32)
        plsc.addupdate_scatter(hist2d, [iota * 256 + d], one16)
        return 0
    lax.fori_loop(0, VPT, hist_body, 0)

    for c in range(16):
        def red_body(r, acc):
            row = hist2d[pl.ds(r * 256 + 16 * c, 16)]
            hist2d[pl.ds(r * 256 + 16 * c, 16)] = z16
            return acc + row
        hist[pl.ds(16 * c, 16)] = lax.fori_loop(0, 16, red_body, z16)

    pltpu.sync_copy(hist, sh_hists.at[pl.ds(tid * 256, 256)])
    plsc.subcore_barrier()
    rds = [pltpu.async_copy(
        sh_hists.at[pl.ds(t * 256 + 16 * tid, 16)],
        blk.at[pl.ds(16 * t, 16)], sem) for t in range(NS)]
    for dsc in rds:
        dsc.wait()

    def pre_body(t, run):
        row = blk[pl.ds(16 * t, 16)]
        preblk[pl.ds(16 * t, 16)] = run
        return run + row
    tot = lax.fori_loop(0, NS, pre_body, z16)
    tmp16[...] = tot
    wds = [pltpu.async_copy(
        preblk.at[pl.ds(16 * t, 16)],
        sh_pre.at[pl.ds(t * 256 + 16 * tid, 16)], sem) for t in range(NS)]
    wds.append(pltpu.async_copy(tmp16, sh_tot.at[pl.ds(16 * tid, 16)], sem))
    for dsc in wds:
        dsc.wait()
    plsc.subcore_barrier()
    d0 = pltpu.async_copy(sh_tot, totv, sem)
    d1 = pltpu.async_copy(sh_pre.at[pl.ds(tid * 256, 256)], prev_own, sem)
    d0.wait()
    d1.wait()

    carry = jnp.int32(0)
    for c in range(16):
        tot_c = totv[pl.ds(16 * c, 16)]
        excl = plsc.cumsum(tot_c) - tot_c
        startv[pl.ds(16 * c, 16)] = excl + carry + prev_own[pl.ds(16 * c, 16)]
        carry = carry + jnp.sum(tot_c)

    for c in range(16):
        countv[pl.ds(16 * c, 16)] = z16

    def rank_body(i, _):
        k = keys_l[pl.ds(16 * i, 16)]
        d = ((k >> sh) & 255).astype(jnp.int32)
        skey = d * 16 + iota
        srt = lax.sort(skey, dimension=0)
        d_s = srt >> 4
        lane_s = srt & 15
        prev = _vgather16(d_s, jnp.maximum(iota - 1, 0))
        nxt = _vgather16(d_s, jnp.minimum(iota + 1, 15))
        newgrp = (iota == 0) | (d_s != prev)
        r = iota - plsc.cummax(jnp.where(newgrp, iota, 0))
        oldc = plsc.load_gather(countv, [d_s])
        bases = plsc.load_gather(startv, [d_s])
        pos = bases + oldc + r
        is_last = (iota == 15) | (d_s != nxt)
        plsc.store_scatter(countv, [d_s], oldc + r + 1, mask=is_last)
        plsc.store_scatter(tmp16, [lane_s], pos)
        dest2d[i // 8, pl.ds(16 * (i % 8), 16)] = tmp16[...]
        return 0
    lax.fori_loop(0, VPT, rank_body, 0)

    descs = []
    for j in range(8):
        idx_row = dest2d.at[j]
        if dst_k is not None:
            descs.append(pltpu.async_copy(
                keys_l.at[pl.ds(128 * j, 128)], dst_k.at[idx_row], sem))
        descs.append(pltpu.async_copy(
            vals_l.at[pl.ds(128 * j, 128)], dst_v.at[idx_row], sem))
    for dsc in descs:
        dsc.wait()
    plsc.subcore_barrier()


def _make_sort_gather():
    mesh = plsc.VectorSubcoreMesh(core_axis_name="c", subcore_axis_name="s")

    @functools.partial(
        pl.kernel,
        out_type=jax.ShapeDtypeStruct((K, 128), jnp.float32),
        mesh=mesh,
        compiler_params=pltpu.CompilerParams(needs_layout_passes=False),
        scratch_types=dict(
            keys_l=pltpu.VMEM((C,), jnp.uint32),
            vals_l=pltpu.VMEM((C,), jnp.int32),
            hist2d=pltpu.VMEM((4096,), jnp.int32),
            hist=pltpu.VMEM((256,), jnp.int32),
            blk=pltpu.VMEM((256,), jnp.int32),
            preblk=pltpu.VMEM((256,), jnp.int32),
            totv=pltpu.VMEM((256,), jnp.int32),
            prev_own=pltpu.VMEM((256,), jnp.int32),
            startv=pltpu.VMEM((256,), jnp.int32),
            countv=pltpu.VMEM((256,), jnp.int32),
            tmp16=pltpu.VMEM((16,), jnp.int32),
            dest2d=pltpu.VMEM((8, 128), jnp.int32),
            idx2d=pltpu.VMEM((2, 128), jnp.int32),
            rows_v=pltpu.VMEM((ROWS_W, 128), jnp.float32),
            a_keys=pltpu.VMEM_SHARED((N,), jnp.uint32),
            a_vals=pltpu.VMEM_SHARED((N,), jnp.int32),
            b_keys=pltpu.VMEM_SHARED((N,), jnp.uint32),
            b_vals=pltpu.VMEM_SHARED((N,), jnp.int32),
            sh_hists=pltpu.VMEM_SHARED((NS * 256,), jnp.int32),
            sh_pre=pltpu.VMEM_SHARED((NS * 256,), jnp.int32),
            sh_tot=pltpu.VMEM_SHARED((256,), jnp.int32),
            sem=pltpu.SemaphoreType.DMA,
        ),
    )
    def sort_gather(keys_hbm, y_hbm, out_hbm, keys_l, vals_l, hist2d, hist,
                    blk, preblk, totv, prev_own, startv, countv, tmp16,
                    dest2d, idx2d, rows_v, a_keys, a_vals, b_keys, b_vals,
                    sh_hists, sh_pre, sh_tot, sem):
        cid = lax.axis_index("c")
        tid = lax.axis_index("s")
        z16 = jnp.zeros((16,), jnp.int32)

        def zero_body(i, _):
            hist2d[pl.ds(16 * i, 16)] = z16
            return 0
        lax.fori_loop(0, 256, zero_body, 0)

        common = (keys_l, vals_l, hist2d, hist, blk, preblk, totv, prev_own,
                  startv, countv, tmp16, dest2d, sh_hists, sh_pre, sh_tot,
                  sem)
        _radix_pass(tid, 0, keys_hbm, None, None, b_keys, b_vals, *common)
        _radix_pass(tid, 8, keys_hbm, b_keys, b_vals, a_keys, a_vals, *common)
        _radix_pass(tid, 16, keys_hbm, a_keys, a_vals, b_keys, b_vals, *common)
        _radix_pass(tid, 24, keys_hbm, b_keys, b_vals, None, a_vals, *common)

        w = tid * NC + cid
        base = w * ROWS_W
        pltpu.sync_copy(a_vals.at[pl.ds(base, 128)], idx2d.at[0])
        pltpu.sync_copy(a_vals.at[pl.ds(base + 128, 128)], idx2d.at[1])
        g0 = pltpu.async_copy(y_hbm.at[idx2d.at[0]],
                              rows_v.at[pl.ds(0, 128)], sem)
        g1 = pltpu.async_copy(y_hbm.at[idx2d.at[1]],
                              rows_v.at[pl.ds(128, 128)], sem)
        g0.wait()
        g1.wait()
        pltpu.sync_copy(rows_v, out_hbm.at[pl.ds(base, ROWS_W)])

    return sort_gather


_SORT_GATHER = _make_sort_gather()


def kernel(x, feature, y, weight_phy, weight_gen, w_phy, w_gen):
    keys, y_pad = _scores_to_keys(feature, weight_phy, weight_gen, w_phy,
                                  w_gen, y)
    y_sel = _SORT_GATHER(keys, y_pad)[:, :DT]
    return (x, feature, y_sel)
